# Initial kernel scaffold; baseline (speedup 1.0000x reference)
import jax
import jax.numpy as jnp
from jax.experimental import pallas as pl


def kernel(x, emb_weight, lin_weight):
    raise NotImplementedError("write your pallas kernel here")


# speedup vs baseline: 3.4163x; 3.4163x over previous
import functools

import jax
import jax.numpy as jnp
from jax import lax
from jax.experimental import pallas as pl
from jax.experimental.pallas import tpu as pltpu
from jax.experimental.pallas import tpu_sc as plsc

_F = 26
_D = 64
_B = 4096
_NPAIR = (_F * (_F - 1)) // 2
_OUT_W = _NPAIR + _F

_PAIRS = [(i, j) for i in range(_F - 1) for j in range(i + 1, _F)]

_NC, _NS = 2, 16
_NW = _NC * _NS
_TOTAL = _B * _F
_PER_W = _TOTAL // _NW
_CHUNK = 832
_NCHUNK = _PER_W // _CHUNK


def _sc_gather_body(x_hbm, emb_hbm, lin_hbm, out_e, out_l,
                    idx_v, rows_v, lin_v, sem):
    wid = lax.axis_index("s") * _NC + lax.axis_index("c")
    base = wid * _PER_W
    pltpu.sync_copy(x_hbm.at[pl.ds(base, _PER_W)], idx_v)
    pltpu.async_copy(lin_hbm.at[idx_v], lin_v, sem).wait()
    pltpu.sync_copy(lin_v, out_l.at[pl.ds(base, _PER_W)])
    for c in range(_NCHUNK):
        pltpu.async_copy(
            emb_hbm.at[idx_v.at[pl.ds(c * _CHUNK, _CHUNK)]], rows_v, sem
        ).wait()
        pltpu.sync_copy(rows_v, out_e.at[pl.ds(base + c * _CHUNK, _CHUNK)])


@functools.cache
def _make_gather():
    return pl.kernel(
        _sc_gather_body,
        out_type=[
            jax.ShapeDtypeStruct((_TOTAL, _D), jnp.float32),
            jax.ShapeDtypeStruct((_TOTAL,), jnp.float32),
        ],
        mesh=plsc.VectorSubcoreMesh(
            core_axis_name="c", subcore_axis_name="s",
            num_cores=_NC, num_subcores=_NS,
        ),
        scratch_types=[
            pltpu.VMEM((_PER_W,), jnp.int32),
            pltpu.VMEM((_CHUNK, _D), jnp.float32),
            pltpu.VMEM((_PER_W,), jnp.float32),
            pltpu.SemaphoreType.DMA,
        ],
        compiler_params=pltpu.CompilerParams(use_tc_tiling_on_sc=False),
    )

_BBLK = 256
_SPAD = 384


def _tc_pairs_body(e_ref, lin_ref, out_ref, s_ref):
    e = e_ref[...]
    et = jnp.transpose(e)
    for p, (i, j) in enumerate(_PAIRS):
        prod = et[i * _D:(i + 1) * _D, :] * et[j * _D:(j + 1) * _D, :]
        s_ref[p, :] = jnp.sum(prod, axis=0)
    ip = jnp.transpose(s_ref[...])[:, :_NPAIR]
    out_ref[...] = jnp.concatenate([ip, lin_ref[...]], axis=1)


_pairs_call = pl.pallas_call(
    _tc_pairs_body,
    out_shape=jax.ShapeDtypeStruct((_B, _OUT_W), jnp.float32),
    grid=(_B // _BBLK,),
    in_specs=[
        pl.BlockSpec((_BBLK, _F * _D), lambda b: (b, 0)),
        pl.BlockSpec((_BBLK, _F), lambda b: (b, 0)),
    ],
    out_specs=pl.BlockSpec((_BBLK, _OUT_W), lambda b: (b, 0)),
    scratch_shapes=[pltpu.VMEM((_SPAD, _BBLK), jnp.float32)],
)


def kernel(x, emb_weight, lin_weight):
    xf = x.reshape(_TOTAL)
    e_g, l_g = _make_gather()(xf, emb_weight, lin_weight.reshape(-1))
    return _pairs_call(e_g.reshape(_B, _F * _D), l_g.reshape(_B, _F))


# speedup vs baseline: 3.5996x; 1.0537x over previous
import functools

import jax
import jax.numpy as jnp
from jax import lax
from jax.experimental import pallas as pl
from jax.experimental.pallas import tpu as pltpu
from jax.experimental.pallas import tpu_sc as plsc

_F = 26
_D = 64
_B = 4096
_NPAIR = (_F * (_F - 1)) // 2
_OUT_W = _NPAIR + _F

_PAIRS = [(i, j) for i in range(_F - 1) for j in range(i + 1, _F)]

_NC, _NS = 2, 16
_NW = _NC * _NS
_TOTAL = _B * _F
_PER_W = _TOTAL // _NW
_CHUNK = 832
_NCHUNK = _PER_W // _CHUNK


def _sc_gather_body(xf_hbm, xtf_hbm, emb_hbm, lin_hbm, out_e, out_l,
                    idx_v, idxt_v, rows_v0, rows_v1, lin_v, gsem, osem):
    wid = lax.axis_index("s") * _NC + lax.axis_index("c")
    base = wid * _PER_W
    pltpu.sync_copy(xf_hbm.at[pl.ds(base, _PER_W)], idx_v)
    pltpu.sync_copy(xtf_hbm.at[pl.ds(base, _PER_W)], idxt_v)
    lin_g = pltpu.async_copy(lin_hbm.at[idxt_v], lin_v, gsem)
    bufs = (rows_v0, rows_v1)
    outs = []
    for c in range(_NCHUNK):
        buf = bufs[c % 2]
        g = pltpu.async_copy(
            emb_hbm.at[idx_v.at[pl.ds(c * _CHUNK, _CHUNK)]], buf, gsem
        )
        if c >= 2:
            outs[c - 2].wait()
        g.wait()
        outs.append(pltpu.async_copy(
            buf, out_e.at[pl.ds(base + c * _CHUNK, _CHUNK)], osem))
    lin_g.wait()
    outs[-2].wait()
    outs[-1].wait()
    pltpu.sync_copy(lin_v, out_l.at[pl.ds(base, _PER_W)])


@functools.cache
def _make_gather():
    return pl.kernel(
        _sc_gather_body,
        out_type=[
            jax.ShapeDtypeStruct((_TOTAL, _D), jnp.float32),
            jax.ShapeDtypeStruct((_TOTAL,), jnp.float32),
        ],
        mesh=plsc.VectorSubcoreMesh(
            core_axis_name="c", subcore_axis_name="s",
            num_cores=_NC, num_subcores=_NS,
        ),
        scratch_types=[
            pltpu.VMEM((_PER_W,), jnp.int32),
            pltpu.VMEM((_PER_W,), jnp.int32),
            pltpu.VMEM((_CHUNK, _D), jnp.float32),
            pltpu.VMEM((_CHUNK, _D), jnp.float32),
            pltpu.VMEM((_PER_W,), jnp.float32),
            pltpu.SemaphoreType.DMA,
            pltpu.SemaphoreType.DMA,
        ],
        compiler_params=pltpu.CompilerParams(use_tc_tiling_on_sc=False),
    )


_BBLK = 256


def _tc_pairs_body(e_ref, lin_ref, out_ref):
    e = e_ref[...]
    et = jnp.transpose(e)
    for p, (i, j) in enumerate(_PAIRS):
        prod = et[i * _D:(i + 1) * _D, :] * et[j * _D:(j + 1) * _D, :]
        out_ref[p, :] = jnp.sum(prod, axis=0)
    out_ref[_NPAIR:_OUT_W, :] = lin_ref[...]


_pairs_call = pl.pallas_call(
    _tc_pairs_body,
    out_shape=jax.ShapeDtypeStruct((_OUT_W, _B), jnp.float32),
    grid=(_B // _BBLK,),
    in_specs=[
        pl.BlockSpec((_BBLK, _F * _D), lambda b: (b, 0)),
        pl.BlockSpec((_F, _BBLK), lambda b: (0, b)),
    ],
    out_specs=pl.BlockSpec((_OUT_W, _BBLK), lambda b: (0, b)),
)


def kernel(x, emb_weight, lin_weight):
    xf = x.reshape(_TOTAL)
    xtf = x.T.reshape(_TOTAL)
    e_g, l_g = _make_gather()(xf, xtf, emb_weight, lin_weight.reshape(-1))
    out_t = _pairs_call(e_g.reshape(_B, _F * _D), l_g.reshape(_F, _B))
    return out_t.T


# speedup vs baseline: 4.0800x; 1.1335x over previous
import functools

import jax
import jax.numpy as jnp
import numpy as np
from jax import lax
from jax.experimental import pallas as pl
from jax.experimental.pallas import tpu as pltpu
from jax.experimental.pallas import tpu_sc as plsc

_F = 26
_D = 64
_B = 4096
_NPAIR = (_F * (_F - 1)) // 2
_OUT_W = _NPAIR + _F

_PAIRS = [(i, j) for i in range(_F - 1) for j in range(i + 1, _F)]

_NC, _NS = 2, 16
_NW = _NC * _NS
_TOTAL = _B * _F
_PER_W = _TOTAL // _NW
_CHUNK = 832
_NCHUNK = _PER_W // _CHUNK

_s = np.arange(_TOTAL, dtype=np.int64)
_b, _f = _s // _F, _s % _F
_DPERM = (((_b // 8) * (_F // 2) + _f // 2) * 16 + (_b % 8) * 2 + (_f % 2))
_DPERM = _DPERM.astype(np.int32).reshape(_NW, _NCHUNK, _CHUNK)


def _sc_gather_body(xf_hbm, xtf_hbm, emb_hbm, lin_hbm, dperm_hbm,
                    out_e, out_l,
                    idx_v, idxt_v, dp_v, rows_v0, rows_v1, lin_v,
                    gsem, osem):
    wid = lax.axis_index("s") * _NC + lax.axis_index("c")
    base = wid * _PER_W
    pltpu.sync_copy(xf_hbm.at[pl.ds(base, _PER_W)], idx_v)
    pltpu.sync_copy(xtf_hbm.at[pl.ds(base, _PER_W)], idxt_v)
    pltpu.sync_copy(dperm_hbm.at[wid], dp_v)
    lin_g = pltpu.async_copy(lin_hbm.at[idxt_v], lin_v, gsem)
    bufs = (rows_v0, rows_v1)
    outs = []
    for c in range(_NCHUNK):
        buf = bufs[c % 2]
        g = pltpu.async_copy(
            emb_hbm.at[idx_v.at[pl.ds(c * _CHUNK, _CHUNK)]], buf, gsem
        )
        if c >= 2:
            outs[c - 2].wait()
        g.wait()
        outs.append(pltpu.async_copy(buf, out_e.at[dp_v.at[c]], osem))
    lin_g.wait()
    outs[-2].wait()
    outs[-1].wait()
    pltpu.sync_copy(lin_v, out_l.at[pl.ds(base, _PER_W)])


@functools.cache
def _make_gather():
    return pl.kernel(
        _sc_gather_body,
        out_type=[
            jax.ShapeDtypeStruct((_TOTAL, _D), jnp.float32),
            jax.ShapeDtypeStruct((_TOTAL,), jnp.float32),
        ],
        mesh=plsc.VectorSubcoreMesh(
            core_axis_name="c", subcore_axis_name="s",
            num_cores=_NC, num_subcores=_NS,
        ),
        scratch_types=[
            pltpu.VMEM((_PER_W,), jnp.int32),
            pltpu.VMEM((_PER_W,), jnp.int32),
            pltpu.VMEM((_NCHUNK, _CHUNK), jnp.int32),
            pltpu.VMEM((_CHUNK, _D), jnp.float32),
            pltpu.VMEM((_CHUNK, _D), jnp.float32),
            pltpu.VMEM((_PER_W,), jnp.float32),
            pltpu.SemaphoreType.DMA,
            pltpu.SemaphoreType.DMA,
        ],
        compiler_params=pltpu.CompilerParams(use_tc_tiling_on_sc=False),
    )


_BBLK = 256
_NBAND = _BBLK // 8


def _tc_pairs_body(e_ref, lin_ref, out_ref):
    e4 = e_ref[...]
    e = jnp.transpose(e4, (0, 2, 1, 3)).reshape(_BBLK, _F * _D)
    et = jnp.transpose(e)
    for p, (i, j) in enumerate(_PAIRS):
        prod = et[i * _D:(i + 1) * _D, :] * et[j * _D:(j + 1) * _D, :]
        out_ref[p, :] = jnp.sum(prod, axis=0)
    out_ref[_NPAIR:_OUT_W, :] = lin_ref[...]


_pairs_call = pl.pallas_call(
    _tc_pairs_body,
    out_shape=jax.ShapeDtypeStruct((_OUT_W, _B), jnp.float32),
    grid=(_B // _BBLK,),
    in_specs=[
        pl.BlockSpec((_NBAND, _F // 2, 8, 128), lambda b: (b, 0, 0, 0)),
        pl.BlockSpec((_F, _BBLK), lambda b: (0, b)),
    ],
    out_specs=pl.BlockSpec((_OUT_W, _BBLK), lambda b: (0, b)),
)


def kernel(x, emb_weight, lin_weight):
    xf = x.reshape(_TOTAL)
    xtf = x.T.reshape(_TOTAL)
    dperm = jnp.asarray(_DPERM)
    e_g, l_g = _make_gather()(xf, xtf, emb_weight, lin_weight.reshape(-1),
                              dperm)
    e4 = e_g.reshape(_B // 8, _F // 2, 8, 128)
    out_t = _pairs_call(e4, l_g.reshape(_F, _B))
    return out_t.T
